# baseline (speedup 1.0000x reference)
import functools

import jax
import jax.numpy as jnp
from jax import lax
from jax.experimental import pallas as pl
from jax.experimental.pallas import tpu as pltpu
from jax.experimental.pallas import tpu_sc as plsc

L = 16
NC = 2
NS = 16
NW = NC * NS
ROWS = 16384
COLS = 200

SC_COLS = 8192
TC_COLS = ROWS - SC_COLS
RPW = SC_COLS // NW
TC_BLK = 512


def _sc_body(idxT_hbm, es_hbm, out_hbm, buf, es_v, out_v, sem):
    wid = lax.axis_index("s") * NC + lax.axis_index("c")
    col_base = wid * RPW

    pltpu.sync_copy(es_hbm, es_v.at[pl.ds(0, 4)])
    ev = es_v[pl.ds(0, L)]
    e_splat = [jnp.full((L,), ev[k], jnp.float32) for k in range(4)]

    pltpu.async_copy(
        idxT_hbm.at[:, pl.ds(col_base, RPW)], buf, sem
    ).wait()

    def group_step(g, _):
        col0 = g * L
        acc = jnp.zeros((L,), jnp.int32)
        for r in range(COLS):
            x = buf[r, pl.ds(col0, L)]
            acc = acc + jnp.left_shift(1, jnp.left_shift(x, 3))

        c0 = jnp.bitwise_and(acc, 255)
        c1 = jnp.bitwise_and(lax.shift_right_logical(acc, 8), 255)
        c2 = jnp.bitwise_and(lax.shift_right_logical(acc, 16), 255)
        c3 = lax.shift_right_logical(acc, 24)
        energy = (
            c0.astype(jnp.float32) * e_splat[0]
            + c1.astype(jnp.float32) * e_splat[1]
            + c2.astype(jnp.float32) * e_splat[2]
            + c3.astype(jnp.float32) * e_splat[3]
        )
        out_v[pl.ds(g * L, L)] = energy
        return 0

    lax.fori_loop(0, RPW // L, group_step, 0)
    pltpu.sync_copy(out_v, out_hbm.at[pl.ds(col_base, RPW)])


@functools.partial(
    pl.kernel,
    out_type=jax.ShapeDtypeStruct((SC_COLS,), jnp.float32),
    mesh=plsc.VectorSubcoreMesh(core_axis_name="c", subcore_axis_name="s"),
    compiler_params=pltpu.CompilerParams(needs_layout_passes=False),
    scratch_types=[
        pltpu.VMEM((COLS, RPW), jnp.int32),
        pltpu.VMEM((L,), jnp.float32),
        pltpu.VMEM((RPW,), jnp.float32),
        pltpu.SemaphoreType.DMA,
    ],
)
def _sc_energy_adder(idxT_hbm, es_hbm, out_hbm, buf, es_v, out_v, sem):
    _sc_body(idxT_hbm, es_hbm, out_hbm, buf, es_v, out_v, sem)


def _tc_kernel(es_ref, x_ref, o_ref):
    x = x_ref[...]
    e = jnp.where(
        x == 0,
        es_ref[0],
        jnp.where(x == 1, es_ref[1], jnp.where(x == 2, es_ref[2], es_ref[3])),
    ).astype(jnp.float32)
    o_ref[...] = jnp.sum(e, axis=0)


_tc_energy_adder = pl.pallas_call(
    _tc_kernel,
    grid=(TC_COLS // TC_BLK,),
    in_specs=[
        pl.BlockSpec(memory_space=pltpu.SMEM),
        pl.BlockSpec((COLS, TC_BLK), lambda i: (0, i)),
    ],
    out_specs=pl.BlockSpec((TC_BLK,), lambda i: (i,)),
    out_shape=jax.ShapeDtypeStruct((TC_COLS,), jnp.float32),
    compiler_params=pltpu.CompilerParams(
        dimension_semantics=("arbitrary",),
    ),
)


def kernel(element_idxs, self_energies):
    idxT = element_idxs.T
    sc_out = _sc_energy_adder(idxT, self_energies)
    tc_out = _tc_energy_adder(
        self_energies, lax.slice(idxT, (0, SC_COLS), (COLS, ROWS))
    )
    return jnp.concatenate([sc_out, tc_out])


# speedup vs baseline: 1.1727x; 1.1727x over previous
import functools

import jax
import jax.numpy as jnp
from jax import lax
from jax.experimental import pallas as pl
from jax.experimental.pallas import tpu as pltpu
from jax.experimental.pallas import tpu_sc as plsc

L = 16
NC = 2
NS = 16
NW = NC * NS
ROWS = 16384
COLS = 200
RPW = ROWS // NW
CCH = 256
NCHUNK = RPW // CCH


def _compute_chunk(buf, out_v, e_splat, out_base):
    def group_step(g, _):
        col0 = g * L
        acc = jnp.zeros((L,), jnp.int32)
        for r in range(COLS):
            x = buf[r, pl.ds(col0, L)]
            acc = acc + jnp.left_shift(1, jnp.left_shift(x, 3))

        c0 = jnp.bitwise_and(acc, 255)
        c1 = jnp.bitwise_and(lax.shift_right_logical(acc, 8), 255)
        c2 = jnp.bitwise_and(lax.shift_right_logical(acc, 16), 255)
        c3 = lax.shift_right_logical(acc, 24)
        energy = (
            c0.astype(jnp.float32) * e_splat[0]
            + c1.astype(jnp.float32) * e_splat[1]
            + c2.astype(jnp.float32) * e_splat[2]
            + c3.astype(jnp.float32) * e_splat[3]
        )
        out_v[pl.ds(out_base + g * L, L)] = energy
        return 0

    lax.fori_loop(0, CCH // L, group_step, 0)


def _body(idxT_hbm, es_hbm, out_hbm, buf0, buf1, es_v, out_v, s0, s1):
    wid = lax.axis_index("s") * NC + lax.axis_index("c")
    col_base = wid * RPW

    pltpu.sync_copy(es_hbm, es_v.at[pl.ds(0, 4)])
    ev = es_v[pl.ds(0, L)]
    e_splat = [jnp.full((L,), ev[k], jnp.float32) for k in range(4)]

    bufs = (buf0, buf1)
    sems = (s0, s1)

    def stage(c, b):
        return pltpu.async_copy(
            idxT_hbm.at[:, pl.ds(col_base + c * CCH, CCH)],
            bufs[b],
            sems[b],
        )

    pending = [None, None]
    pending[0] = stage(0, 0)
    for c in range(NCHUNK):
        b = c % 2
        if c + 1 < NCHUNK:
            pending[1 - b] = stage(c + 1, 1 - b)
        pending[b].wait()
        _compute_chunk(bufs[b], out_v, e_splat, c * CCH)

    pltpu.sync_copy(out_v, out_hbm.at[pl.ds(wid * RPW, RPW)])


@functools.partial(
    pl.kernel,
    out_type=jax.ShapeDtypeStruct((ROWS,), jnp.float32),
    mesh=plsc.VectorSubcoreMesh(core_axis_name="c", subcore_axis_name="s"),
    compiler_params=pltpu.CompilerParams(needs_layout_passes=False),
    scratch_types=[
        pltpu.VMEM((COLS, CCH), jnp.int32),
        pltpu.VMEM((COLS, CCH), jnp.int32),
        pltpu.VMEM((L,), jnp.float32),
        pltpu.VMEM((RPW,), jnp.float32),
        pltpu.SemaphoreType.DMA,
        pltpu.SemaphoreType.DMA,
    ],
)
def _energy_adder(idxT_hbm, es_hbm, out_hbm, buf0, buf1, es_v, out_v, s0, s1):
    _body(idxT_hbm, es_hbm, out_hbm, buf0, buf1, es_v, out_v, s0, s1)


def kernel(element_idxs, self_energies):
    return _energy_adder(element_idxs.T, self_energies)
